# baseline (speedup 1.0000x reference)
import jax
import jax.numpy as jnp
from jax import lax
from jax.experimental import pallas as pl
from jax.experimental.pallas import tpu as pltpu

_E = 64
_K = 8
_BT = 1024
_KC = 2048


def _router_body(x_ref, w_ref, scores_ref, topw_ref, topi_ref, acc_ref):
    j = pl.program_id(1)
    xb = x_ref[...]
    wf = w_ref[j]
    partial = lax.dot_general(
        xb, wf, (((1,), (1,)), ((), ())),
        preferred_element_type=jnp.float32)

    @pl.when(j == 0)
    def _():
        acc_ref[...] = partial

    @pl.when(j == 1)
    def _():
        logits = acc_ref[...] + partial
        m = jnp.max(logits, axis=-1, keepdims=True)
        unnorm = jnp.exp(logits - m)
        scores = unnorm / jnp.sum(unnorm, axis=-1, keepdims=True)
        scores_ref[...] = scores

        iota = lax.broadcasted_iota(jnp.int32, scores.shape, 1)
        cur = scores
        ws = []
        idxs = []
        for _ in range(_K):
            mk = jnp.max(cur, axis=-1, keepdims=True)
            hit = cur == mk
            ik = jnp.min(jnp.where(hit, iota, _E), axis=-1, keepdims=True)
            ws.append(mk)
            idxs.append(ik)
            cur = jnp.where(iota == ik, -1.0, cur)
        topw_ref[...] = jnp.concatenate(ws, axis=1)
        topi_ref[...] = jnp.concatenate(idxs, axis=1)


@jax.jit
def kernel(x, W):
    sl, bs, hs = x.shape
    t = sl * bs
    xt = x.reshape(t, hs)
    w_r = W.reshape(_E, 2, _KC).transpose(1, 0, 2)
    grid = (t // _BT, 2)
    scores, topw, topi = pl.pallas_call(
        _router_body,
        grid=grid,
        in_specs=[
            pl.BlockSpec((_BT, _KC), lambda i, j: (i, j)),
            pl.BlockSpec(memory_space=pltpu.VMEM),
        ],
        out_specs=[
            pl.BlockSpec((_BT, _E), lambda i, j: (i, 0)),
            pl.BlockSpec((_BT, _K), lambda i, j: (i, 0)),
            pl.BlockSpec((_BT, _K), lambda i, j: (i, 0)),
        ],
        out_shape=[
            jax.ShapeDtypeStruct((t, _E), jnp.float32),
            jax.ShapeDtypeStruct((t, _K), jnp.float32),
            jax.ShapeDtypeStruct((t, _K), jnp.int32),
        ],
        scratch_shapes=[pltpu.VMEM((_BT, _E), jnp.float32)],
        compiler_params=pltpu.CompilerParams(
            dimension_semantics=("arbitrary", "arbitrary")),
    )(xt, w_r)
    return scores, topw, topi, jnp.float32(0.0)


# speedup vs baseline: 1.0995x; 1.0995x over previous
import jax
import jax.numpy as jnp
from jax import lax
from jax.experimental import pallas as pl
from jax.experimental.pallas import tpu as pltpu

_E = 64
_K = 8
_BT = 1024


def _router_body(x_ref, w_ref, scores_ref, topw_ref, topi_ref):
    xb = x_ref[...]
    wf = w_ref[...]
    logits = lax.dot_general(
        xb, wf, (((1,), (1,)), ((), ())),
        preferred_element_type=jnp.float32)
    m = jnp.max(logits, axis=-1, keepdims=True)
    unnorm = jnp.exp(logits - m)
    scores = unnorm / jnp.sum(unnorm, axis=-1, keepdims=True)
    scores_ref[...] = scores

    iota = lax.broadcasted_iota(jnp.int32, scores.shape, 1)
    cur = scores
    ws = []
    idxs = []
    for _ in range(_K):
        mk = jnp.max(cur, axis=-1, keepdims=True)
        hit = cur == mk
        ik = jnp.min(jnp.where(hit, iota, _E), axis=-1, keepdims=True)
        ws.append(mk)
        idxs.append(ik)
        cur = jnp.where(iota == ik, -1.0, cur)
    topw_ref[...] = jnp.concatenate(ws, axis=1)
    topi_ref[...] = jnp.concatenate(idxs, axis=1)


@jax.jit
def kernel(x, W):
    sl, bs, hs = x.shape
    t = sl * bs
    xt = x.reshape(t, hs)
    grid = (t // _BT,)
    scores, topw, topi = pl.pallas_call(
        _router_body,
        grid=grid,
        in_specs=[
            pl.BlockSpec((_BT, hs), lambda i: (i, 0)),
            pl.BlockSpec(memory_space=pltpu.VMEM),
        ],
        out_specs=[
            pl.BlockSpec((_BT, _E), lambda i: (i, 0)),
            pl.BlockSpec((_BT, _K), lambda i: (i, 0)),
            pl.BlockSpec((_BT, _K), lambda i: (i, 0)),
        ],
        out_shape=[
            jax.ShapeDtypeStruct((t, _E), jnp.float32),
            jax.ShapeDtypeStruct((t, _K), jnp.float32),
            jax.ShapeDtypeStruct((t, _K), jnp.int32),
        ],
        compiler_params=pltpu.CompilerParams(
            dimension_semantics=("arbitrary",)),
    )(xt, W)
    return scores, topw, topi, jnp.float32(0.0)


# speedup vs baseline: 1.1304x; 1.0282x over previous
import jax
import jax.numpy as jnp
from jax import lax
from jax.experimental import pallas as pl
from jax.experimental.pallas import tpu as pltpu

_E = 64
_K = 8
_BT = 1024


def _router_body(x_ref, w_ref, scores_ref, topw_ref, topi_ref):
    xb = x_ref[...]
    wf = w_ref[...]
    logits = lax.dot_general(
        xb, wf, (((1,), (1,)), ((), ())),
        preferred_element_type=jnp.float32)
    m = jnp.max(logits, axis=-1, keepdims=True)
    unnorm = jnp.exp(logits - m)
    scores = unnorm / jnp.sum(unnorm, axis=-1, keepdims=True)
    scores_ref[...] = scores

    iota = lax.broadcasted_iota(jnp.int32, scores.shape, 1)
    cur = scores
    ws = []
    idxs = []
    for _ in range(_K):
        mk = jnp.max(cur, axis=-1, keepdims=True)
        hit = cur == mk
        ik = jnp.min(jnp.where(hit, iota, _E), axis=-1, keepdims=True)
        ws.append(mk)
        idxs.append(ik)
        cur = jnp.where(iota == ik, -1.0, cur)
    topw_ref[...] = jnp.concatenate(ws, axis=1).T
    topi_ref[...] = jnp.concatenate(idxs, axis=1).T


@jax.jit
def kernel(x, W):
    sl, bs, hs = x.shape
    t = sl * bs
    xt = x.reshape(t, hs)
    grid = (t // _BT,)
    scores, topw, topi = pl.pallas_call(
        _router_body,
        grid=grid,
        in_specs=[
            pl.BlockSpec((_BT, hs), lambda i: (i, 0)),
            pl.BlockSpec(memory_space=pltpu.VMEM),
        ],
        out_specs=[
            pl.BlockSpec((_BT, _E), lambda i: (i, 0)),
            pl.BlockSpec((_K, _BT), lambda i: (0, i)),
            pl.BlockSpec((_K, _BT), lambda i: (0, i)),
        ],
        out_shape=[
            jax.ShapeDtypeStruct((t, _E), jnp.float32),
            jax.ShapeDtypeStruct((_K, t), jnp.float32),
            jax.ShapeDtypeStruct((_K, t), jnp.int32),
        ],
        compiler_params=pltpu.CompilerParams(
            dimension_semantics=("arbitrary",)),
    )(xt, W)
    return scores, topw.T, topi.T, jnp.float32(0.0)
